# baseline (speedup 1.0000x reference)
import functools
import math

import jax
import jax.numpy as jnp
from jax import lax
from jax.experimental import pallas as pl
from jax.experimental.pallas import tpu as pltpu
from jax.experimental.pallas import tpu_sc as plsc

B = 4096
L = 200
EMB = 128
NGRAM = 3
DOUT = EMB * NGRAM
HALF = L // 2

NC = 2
NS = 16
NL = 16
NB = 32
NW = NC * NS
RPW = B // NW
HALFROWS = RPW // 2
IDXR = RPW * 2

S2 = math.sqrt(float(EMB)) / 2.0
S3 = float(EMB) / 3.0
K = 8
GPP = 2
NPASS = EMB // (NB * GPP)


def _disc_body(tok_hbm, table_hbm, out_hbm, idx_v, emb0, emb1, out_v, sem0, sem1):
  wid = lax.axis_index("s") * NC + lax.axis_index("c")

  pltpu.sync_copy(tok_hbm.at[pl.ds(wid * IDXR, IDXR)], idx_v)

  def start_gather(r, emb, sem):
    j0 = 2 * r
    pltpu.async_copy(table_hbm.at[idx_v.at[j0]], emb.at[pl.ds(0, HALF)], sem)
    pltpu.async_copy(table_hbm.at[idx_v.at[j0 + 1]], emb.at[pl.ds(HALF, HALF)], sem)

  def wait_gather(emb, sem):
    pltpu.make_async_copy(table_hbm.at[pl.ds(0, L)], emb, sem).wait()

  def unpack_even_odd(x):
    w = plsc.bitcast(x, jnp.uint32)
    even = plsc.bitcast(jnp.left_shift(w, 16), jnp.float32)
    odd = plsc.bitcast(w & jnp.full((NL,), 0xFFFF0000, jnp.uint32), jnp.float32)
    return even, odd

  def compute_row(r, emb):
    slot = lax.rem(r, HALFROWS)
    zf = jnp.zeros((NL,), jnp.float32)
    zb = jnp.zeros((NB,), jnp.bfloat16)
    for base in range(0, EMB // NB, GPP):

      @plsc.parallel_loop(0, L // K, carry=(zf,) * (6 * GPP) + (zb,) * (2 * GPP))
      def _loop(blk, carry):
        i0 = blk * K
        accs = list(carry[:6 * GPP])
        prevs = list(carry[6 * GPP:])
        for g in range(GPP):
          p1, q1 = prevs[2 * g], prevs[2 * g + 1]
          b1 = b2 = b3 = zb
          for j in range(K):
            v = emb[i0 + j, pl.ds((base + g) * NB, NB)]
            p = v * p1
            b1 = b1 + v
            b2 = b2 + p
            b3 = b3 + v * q1
            p1, q1 = v, p
          for a, blkacc in enumerate((b1, b2, b3)):
            e, o = unpack_even_odd(blkacc)
            accs[6 * g + 2 * a] = accs[6 * g + 2 * a] + e
            accs[6 * g + 2 * a + 1] = accs[6 * g + 2 * a + 1] + o
          prevs[2 * g], prevs[2 * g + 1] = p1, q1
        return tuple(accs) + tuple(prevs)

      rows = jnp.full((NL,), slot, jnp.int32)
      iota2 = lax.iota(jnp.int32, NL) * 2
      for g in range(GPP):
        c0 = (base + g) * NB
        a1e, a1o, a2e, a2o, a3e, a3o = _loop[6 * g:6 * g + 6]
        cols = iota2 + c0
        plsc.store_scatter(out_v, [rows, cols], a1e)
        plsc.store_scatter(out_v, [rows, cols + 1], a1o)
        plsc.store_scatter(out_v, [rows, cols + EMB], a2e * S2)
        plsc.store_scatter(out_v, [rows, cols + (EMB + 1)], a2o * S2)
        plsc.store_scatter(out_v, [rows, cols + 2 * EMB], a3e * S3)
        plsc.store_scatter(out_v, [rows, cols + (2 * EMB + 1)], a3o * S3)

  start_gather(0, emb0, sem0)
  start_gather(1, emb1, sem1)

  def outer(g, carry):
    for b, (emb, sem) in enumerate(((emb0, sem0), (emb1, sem1))):
      r = 2 * g + b
      wait_gather(emb, sem)
      compute_row(r, emb)

      @pl.when(r + 2 < RPW)
      def _():
        start_gather(r + 2, emb, sem)

    @pl.when(g == HALFROWS // 2 - 1)
    def _():
      pltpu.sync_copy(out_v, out_hbm.at[pl.ds(wid * RPW, HALFROWS)])

    return carry

  lax.fori_loop(0, RPW // 2, outer, 0)

  pltpu.sync_copy(out_v, out_hbm.at[pl.ds(wid * RPW + HALFROWS, HALFROWS)])


@functools.partial(
    pl.kernel,
    out_type=jax.ShapeDtypeStruct((B, DOUT), jnp.float32),
    mesh=plsc.VectorSubcoreMesh(core_axis_name="c", subcore_axis_name="s"),
    scratch_types=[
        pltpu.VMEM((IDXR, HALF), jnp.int32),
        pltpu.VMEM((L, EMB), jnp.bfloat16),
        pltpu.VMEM((L, EMB), jnp.bfloat16),
        pltpu.VMEM((RPW // 2, DOUT), jnp.float32),
        pltpu.SemaphoreType.DMA,
        pltpu.SemaphoreType.DMA,
    ],
    compiler_params=pltpu.CompilerParams(use_tc_tiling_on_sc=False, needs_layout_passes=False),
)
def _disc_embed(tok_hbm, table_hbm, out_hbm, idx_v, emb0, emb1, out_v, sem0, sem1):
  _disc_body(tok_hbm, table_hbm, out_hbm, idx_v, emb0, emb1, out_v, sem0, sem1)


def _cast_body(x_ref, o_ref):
  o_ref[...] = x_ref[...].astype(jnp.bfloat16)


_CAST_BLK = 1000


def _cast_table(table):
  vocab = table.shape[0]
  return pl.pallas_call(
      _cast_body,
      out_shape=jax.ShapeDtypeStruct((vocab, EMB), jnp.bfloat16),
      grid=(vocab // _CAST_BLK,),
      in_specs=[pl.BlockSpec((_CAST_BLK, EMB), lambda i: (i, 0))],
      out_specs=pl.BlockSpec((_CAST_BLK, EMB), lambda i: (i, 0)),
  )(table)


def kernel(tokens, table):
  tokens2d = tokens.reshape(NW * IDXR, HALF)
  table_bf = _cast_table(table)
  return _disc_embed(tokens2d, table_bf)


# speedup vs baseline: 1.4357x; 1.4357x over previous
import functools
import math

import jax
import jax.numpy as jnp
from jax import lax
from jax.experimental import pallas as pl
from jax.experimental.pallas import tpu as pltpu
from jax.experimental.pallas import tpu_sc as plsc

B = 4096
L = 200
EMB = 128
NGRAM = 3
DOUT = EMB * NGRAM
HALF = L // 2

NC = 2
NS = 16
NL = 16
NB = 32
NW = NC * NS
RPW = B // NW
HALFROWS = RPW // 2
IDXR = RPW * 2

S2 = math.sqrt(float(EMB)) / 2.0
S3 = float(EMB) / 3.0
K = 8
GPP = 2
NPASS = EMB // (NB * GPP)


def _disc_body(tok_hbm, table_hbm, out_hbm, idx_v, emb0, emb1, out_v, sem0, sem1):
  wid = lax.axis_index("s") * NC + lax.axis_index("c")

  pltpu.sync_copy(tok_hbm.at[pl.ds(wid * IDXR, IDXR)], idx_v)

  def start_gather(r, emb, sem):
    j0 = 2 * r
    pltpu.async_copy(table_hbm.at[idx_v.at[j0]], emb.at[pl.ds(0, HALF)], sem)
    pltpu.async_copy(table_hbm.at[idx_v.at[j0 + 1]], emb.at[pl.ds(HALF, HALF)], sem)

  def wait_gather(emb, sem):
    pltpu.make_async_copy(table_hbm.at[pl.ds(0, L)], emb, sem).wait()

  def unpack_even_odd(x):
    w = plsc.bitcast(x, jnp.uint32)
    even = plsc.bitcast(jnp.left_shift(w, 16), jnp.float32)
    odd = plsc.bitcast(w & jnp.full((NL,), 0xFFFF0000, jnp.uint32), jnp.float32)
    return even, odd

  def compute_row(r, emb):
    slot = lax.rem(r, HALFROWS)
    zf = jnp.zeros((NL,), jnp.float32)
    zb = jnp.zeros((NB,), jnp.bfloat16)
    for base in range(0, EMB // NB, GPP):

      @plsc.parallel_loop(0, L // K, carry=(zf,) * (6 * GPP) + (zb,) * (2 * GPP))
      def _loop(blk, carry):
        i0 = blk * K
        accs = list(carry[:6 * GPP])
        prevs = list(carry[6 * GPP:])
        for g in range(GPP):
          p1, q1 = prevs[2 * g], prevs[2 * g + 1]
          b1 = b2 = b3 = zb
          for j in range(K):
            lo = emb[i0 + j, pl.ds((base + g) * NB, NL)]
            hi = emb[i0 + j, pl.ds((base + g) * NB + NL, NL)]
            v = plsc.pack(lo, hi, format=plsc.PackFormat.INTERLEAVED)
            p = v * p1
            b1 = b1 + v
            b2 = b2 + p
            b3 = b3 + v * q1
            p1, q1 = v, p
          for a, blkacc in enumerate((b1, b2, b3)):
            e, o = unpack_even_odd(blkacc)
            accs[6 * g + 2 * a] = accs[6 * g + 2 * a] + e
            accs[6 * g + 2 * a + 1] = accs[6 * g + 2 * a + 1] + o
          prevs[2 * g], prevs[2 * g + 1] = p1, q1
        return tuple(accs) + tuple(prevs)

      for g in range(GPP):
        c0 = (base + g) * NB
        a1e, a1o, a2e, a2o, a3e, a3o = _loop[6 * g:6 * g + 6]
        out_v[slot, pl.ds(c0, NL)] = a1e
        out_v[slot, pl.ds(c0 + NL, NL)] = a1o
        out_v[slot, pl.ds(EMB + c0, NL)] = a2e * S2
        out_v[slot, pl.ds(EMB + c0 + NL, NL)] = a2o * S2
        out_v[slot, pl.ds(2 * EMB + c0, NL)] = a3e * S3
        out_v[slot, pl.ds(2 * EMB + c0 + NL, NL)] = a3o * S3

  start_gather(0, emb0, sem0)
  start_gather(1, emb1, sem1)

  def outer(g, carry):
    for b, (emb, sem) in enumerate(((emb0, sem0), (emb1, sem1))):
      r = 2 * g + b
      wait_gather(emb, sem)
      compute_row(r, emb)

      @pl.when(r + 2 < RPW)
      def _():
        start_gather(r + 2, emb, sem)

    @pl.when(g == HALFROWS // 2 - 1)
    def _():
      pltpu.sync_copy(out_v, out_hbm.at[pl.ds(wid * RPW, HALFROWS)])

    return carry

  lax.fori_loop(0, RPW // 2, outer, 0)

  pltpu.sync_copy(out_v, out_hbm.at[pl.ds(wid * RPW + HALFROWS, HALFROWS)])


@functools.partial(
    pl.kernel,
    out_type=jax.ShapeDtypeStruct((B, DOUT), jnp.float32),
    mesh=plsc.VectorSubcoreMesh(core_axis_name="c", subcore_axis_name="s"),
    scratch_types=[
        pltpu.VMEM((IDXR, HALF), jnp.int32),
        pltpu.VMEM((L, EMB), jnp.float32),
        pltpu.VMEM((L, EMB), jnp.float32),
        pltpu.VMEM((RPW // 2, DOUT), jnp.float32),
        pltpu.SemaphoreType.DMA,
        pltpu.SemaphoreType.DMA,
    ],
    compiler_params=pltpu.CompilerParams(use_tc_tiling_on_sc=False, needs_layout_passes=False),
)
def _disc_embed(tok_hbm, table_hbm, out_hbm, idx_v, emb0, emb1, out_v, sem0, sem1):
  _disc_body(tok_hbm, table_hbm, out_hbm, idx_v, emb0, emb1, out_v, sem0, sem1)


def kernel(tokens, table):
  tokens2d = tokens.reshape(NW * IDXR, HALF)
  return _disc_embed(tokens2d, table)
